# baseline (speedup 1.0000x reference)
import functools
import numpy as np
import jax
import jax.numpy as jnp
from jax import lax
from jax.experimental import pallas as pl
from jax.experimental.pallas import tpu as pltpu
from jax.experimental.pallas import tpu_sc as plsc

_NWORDS = 8192
_K = 50
_EMB = 64
_ROW = 2 * _EMB
_TP = _NWORDS * _K // 2

_NC, _NS = 2, 16
_NWK = _NC * _NS
_PPW = 512
_CH = 128
_NCH = _PPW // _CH
_SCP = _NWK * _PPW
_SCBASE = _TP - _SCP

_TCB = 512
_TCG = _SCBASE // _TCB

_WEF = (2 * np.arange(_TP, dtype=np.int32)) // _K
_WOF = (2 * np.arange(_TP, dtype=np.int32) + 1) // _K

_mesh = plsc.VectorSubcoreMesh(
    core_axis_name="c", subcore_axis_name="s", num_cores=_NC, num_subcores=_NS
)


def _bucket(word, top):
    d = jnp.maximum(word - top, 1)
    one = jnp.int32(1)
    zero = jnp.int32(0)
    bl = (
        4
        + jnp.where(d >= 8, one, zero)
        + jnp.where(d >= 16, one, zero)
        + jnp.where(d >= 32, one, zero)
        + jnp.where(d >= 64, one, zero)
    )
    return jnp.where(d < 5, d - 1, bl)


def _sc_body(tope_hbm, topo_hbm, we_hbm, wo_hbm, ptab_hbm, out_hbm,
             tope_v, topo_v, we_v, wo_v, ptab_v, ptab_s, idx_v,
             rows0, rows1, gsem, osem0, osem1):
    sid = lax.axis_index("s")
    wid = sid * _NC + lax.axis_index("c")
    pbase = _SCBASE + wid * _PPW
    pltpu.sync_copy(tope_hbm.at[pl.ds(pbase, _PPW)], tope_v)
    pltpu.sync_copy(topo_hbm.at[pl.ds(pbase, _PPW)], topo_v)
    pltpu.sync_copy(we_hbm.at[pl.ds(pbase, _PPW)], we_v)
    pltpu.sync_copy(wo_hbm.at[pl.ds(pbase, _PPW)], wo_v)
    pltpu.sync_copy(ptab_hbm, ptab_v)
    pltpu.sync_copy(ptab_hbm, ptab_s.at[pl.ds(sid * 81, 81)])
    sbase = sid * 81
    lane = lax.iota(jnp.int32, 16)

    def pair_indices(j):
        te = tope_v[pl.ds(j, 16)]
        to = topo_v[pl.ds(j, 16)]
        we = we_v[pl.ds(j, 16)]
        wo = wo_v[pl.ds(j, 16)]
        return _bucket(we, te) * 9 + _bucket(wo, to)

    def compute_idx(c, idxbuf):
        for g in range(_CH // 16):
            idxbuf[pl.ds(g * 16, 16)] = pair_indices(c * _CH + g * 16) + sbase

    def fill(c, rowsbuf):
        def group(g, carry):
            gaddr = pair_indices(c * _CH + g * 16)
            prow = lane + g * 16

            @plsc.parallel_loop(0, _ROW, step=1, unroll=8)
            def colloop(t):
                colv = (t + lane) & (_ROW - 1)
                v = plsc.load_gather(ptab_v, [gaddr, colv])
                plsc.store_scatter(rowsbuf, [prow, colv], v)

            return carry

        lax.fori_loop(0, _CH // 16, group, 0)

    def out_ref(c):
        return out_hbm.at[pl.ds(pbase + c * _CH, _CH)]

    def step(i, carry):
        c0 = 2 * i
        c1 = 2 * i + 1
        compute_idx(c0, idx_v)

        @pl.when(i >= 1)
        def _():
            pltpu.make_async_copy(rows0, out_ref(c0 - 2), osem0).wait()

        pltpu.async_copy(ptab_s.at[idx_v], rows0, gsem)

        @pl.when(i >= 1)
        def _():
            pltpu.make_async_copy(rows1, out_ref(c1 - 2), osem1).wait()

        fill(c1, rows1)
        pltpu.async_copy(rows1, out_ref(c1), osem1)
        pltpu.make_async_copy(ptab_s.at[idx_v], rows0, gsem).wait()
        pltpu.async_copy(rows0, out_ref(c0), osem0)
        return carry

    lax.fori_loop(0, _NCH // 2, step, 0)
    pltpu.make_async_copy(rows0, out_ref(_NCH - 2), osem0).wait()
    pltpu.make_async_copy(rows1, out_ref(_NCH - 1), osem1).wait()


_sc_lookup = pl.kernel(
    _sc_body,
    out_type=jax.ShapeDtypeStruct((_TP, _ROW), jnp.float32),
    mesh=_mesh,
    scratch_types=[
        pltpu.VMEM((_PPW,), jnp.int32),
        pltpu.VMEM((_PPW,), jnp.int32),
        pltpu.VMEM((_PPW,), jnp.int32),
        pltpu.VMEM((_PPW,), jnp.int32),
        pltpu.VMEM((81, _ROW), jnp.float32),
        pltpu.VMEM_SHARED((_NS * 81, _ROW), jnp.float32),
        pltpu.VMEM((_CH,), jnp.int32),
        pltpu.VMEM((_CH, _ROW), jnp.float32),
        pltpu.VMEM((_CH, _ROW), jnp.float32),
        pltpu.SemaphoreType.DMA,
        pltpu.SemaphoreType.DMA,
        pltpu.SemaphoreType.DMA,
    ],
    compiler_params=pltpu.CompilerParams(
        needs_layout_passes=False, disable_bounds_checks=True
    ),
)


def _tc_body(tope_ref, topo_ref, we_ref, wo_ref, tbl_ref, alias_ref, out_ref):
    del alias_ref
    be = _bucket(we_ref[...], tope_ref[...])
    bo = _bucket(wo_ref[...], topo_ref[...])
    i9 = lax.broadcasted_iota(jnp.int32, (_TCB, 9), 1)
    ohe = (be == i9).astype(jnp.float32)
    oho = (bo == i9).astype(jnp.float32)
    oh = jnp.concatenate([ohe, oho], axis=1)
    out_ref[...] = jnp.dot(oh, tbl_ref[...], preferred_element_type=jnp.float32)


def _col_spec():
    return pl.BlockSpec((_TCB, 1), lambda b: (b, 0))


_tc_expand = pl.pallas_call(
    _tc_body,
    grid=(_TCG,),
    in_specs=[
        _col_spec(),
        _col_spec(),
        _col_spec(),
        _col_spec(),
        pl.BlockSpec((18, _ROW), lambda b: (0, 0)),
        pl.BlockSpec(memory_space=pltpu.HBM),
    ],
    out_specs=pl.BlockSpec((_TCB, _ROW), lambda b: (b, 0)),
    out_shape=jax.ShapeDtypeStruct((_TP, _ROW), jnp.float32),
    input_output_aliases={5: 0},
)


@jax.jit
def kernel(top_indices, distance_emb):
    emb = distance_emb.astype(jnp.float32)
    ptab = jnp.concatenate(
        [
            jnp.broadcast_to(emb[:, None, :], (9, 9, _EMB)),
            jnp.broadcast_to(emb[None, :, :], (9, 9, _EMB)),
        ],
        axis=-1,
    ).reshape(81, _ROW)
    z = jnp.zeros((9, _EMB), jnp.float32)
    tbl = jnp.concatenate(
        [
            jnp.concatenate([emb, z], axis=1),
            jnp.concatenate([z, emb], axis=1),
        ],
        axis=0,
    )
    top_flat = top_indices.reshape(-1).astype(jnp.int32)
    tope = top_flat[0::2]
    topo = top_flat[1::2]
    wef = jnp.asarray(_WEF)
    wof = jnp.asarray(_WOF)
    out_sc = _sc_lookup(tope, topo, wef, wof, ptab)
    out = _tc_expand(
        tope.reshape(_TP, 1),
        topo.reshape(_TP, 1),
        wef.reshape(_TP, 1),
        wof.reshape(_TP, 1),
        tbl,
        out_sc,
    )
    return out.reshape(_NWORDS, _K, _EMB)


# speedup vs baseline: 2.0003x; 2.0003x over previous
import functools
import numpy as np
import jax
import jax.numpy as jnp
from jax import lax
from jax.experimental import pallas as pl
from jax.experimental.pallas import tpu as pltpu
from jax.experimental.pallas import tpu_sc as plsc

_NWORDS = 8192
_K = 50
_EMB = 64
_ROW = 2 * _EMB
_TP = _NWORDS * _K // 2

_NC, _NS = 2, 16
_NWK = _NC * _NS
_IPW = _TP // _NWK
_PPW = 512
_CH = 128
_NCH = _PPW // _CH
_SCP = _NWK * _PPW
_SCBASE = _TP - _SCP

_TCB = 1024
_TCG = _SCBASE // _TCB

_WEF = (2 * np.arange(_TP, dtype=np.int32)) // _K
_WOF = (2 * np.arange(_TP, dtype=np.int32) + 1) // _K

_mesh = plsc.VectorSubcoreMesh(
    core_axis_name="c", subcore_axis_name="s", num_cores=_NC, num_subcores=_NS
)


def _bucket(word, top):
    d = jnp.maximum(word - top, 1)
    one = jnp.int32(1)
    zero = jnp.int32(0)
    bl = (
        4
        + jnp.where(d >= 8, one, zero)
        + jnp.where(d >= 16, one, zero)
        + jnp.where(d >= 32, one, zero)
        + jnp.where(d >= 64, one, zero)
    )
    return jnp.where(d < 5, d - 1, bl)


def _sc_body(top_hbm, we_hbm, wo_hbm, ptab_hbm, out_hbm, pidx_hbm,
             top_v, we_v, wo_v, pidx_v, ttop_v, twe_v, two_v,
             ptab_v, ptab_s, idx_v, rows0, rows1, gsem, osem0, osem1):
    sid = lax.axis_index("s")
    wid = sid * _NC + lax.axis_index("c")
    lane = lax.iota(jnp.int32, 16)
    lane2 = 2 * lane

    ibase = wid * _IPW
    pltpu.sync_copy(top_hbm.at[pl.ds(2 * ibase, 2 * _IPW)], top_v)
    pltpu.sync_copy(we_hbm.at[pl.ds(ibase, _IPW)], we_v)
    pltpu.sync_copy(wo_hbm.at[pl.ds(ibase, _IPW)], wo_v)

    def idx_group(g, carry):
        j = g * 16
        te = plsc.load_gather(top_v, [2 * j + lane2])
        to = plsc.load_gather(top_v, [2 * j + lane2 + 1])
        we = we_v[pl.ds(j, 16)]
        wo = wo_v[pl.ds(j, 16)]
        pidx_v[pl.ds(j, 16)] = _bucket(we, te) * 9 + _bucket(wo, to)
        return carry

    lax.fori_loop(0, _IPW // 16, idx_group, 0)
    pltpu.sync_copy(pidx_v, pidx_hbm.at[pl.ds(ibase, _IPW)])

    pbase = _SCBASE + wid * _PPW
    pltpu.sync_copy(top_hbm.at[pl.ds(2 * pbase, 2 * _PPW)], ttop_v)
    pltpu.sync_copy(we_hbm.at[pl.ds(pbase, _PPW)], twe_v)
    pltpu.sync_copy(wo_hbm.at[pl.ds(pbase, _PPW)], two_v)
    pltpu.sync_copy(ptab_hbm, ptab_v)
    pltpu.sync_copy(ptab_hbm, ptab_s.at[pl.ds(sid * 81, 81)])
    sbase = sid * 81

    def pair_indices(j):
        te = plsc.load_gather(ttop_v, [2 * j + lane2])
        to = plsc.load_gather(ttop_v, [2 * j + lane2 + 1])
        we = twe_v[pl.ds(j, 16)]
        wo = two_v[pl.ds(j, 16)]
        return _bucket(we, te) * 9 + _bucket(wo, to)

    def compute_idx(c, idxbuf):
        for g in range(_CH // 16):
            idxbuf[pl.ds(g * 16, 16)] = pair_indices(c * _CH + g * 16) + sbase

    def fill(c, rowsbuf):
        def group(g, carry):
            gaddr = pair_indices(c * _CH + g * 16)
            prow = lane + g * 16

            @plsc.parallel_loop(0, _ROW, step=1, unroll=8)
            def colloop(t):
                colv = (t + lane) & (_ROW - 1)
                v = plsc.load_gather(ptab_v, [gaddr, colv])
                plsc.store_scatter(rowsbuf, [prow, colv], v)

            return carry

        lax.fori_loop(0, _CH // 16, group, 0)

    def out_ref(c):
        return out_hbm.at[pl.ds(pbase + c * _CH, _CH)]

    def step(i, carry):
        c0 = 2 * i
        c1 = 2 * i + 1
        compute_idx(c0, idx_v)

        @pl.when(i >= 1)
        def _():
            pltpu.make_async_copy(rows0, out_ref(c0 - 2), osem0).wait()

        pltpu.async_copy(ptab_s.at[idx_v], rows0, gsem)

        @pl.when(i >= 1)
        def _():
            pltpu.make_async_copy(rows1, out_ref(c1 - 2), osem1).wait()

        fill(c1, rows1)
        pltpu.async_copy(rows1, out_ref(c1), osem1)
        pltpu.make_async_copy(ptab_s.at[idx_v], rows0, gsem).wait()
        pltpu.async_copy(rows0, out_ref(c0), osem0)
        return carry

    lax.fori_loop(0, _NCH // 2, step, 0)
    pltpu.make_async_copy(rows0, out_ref(_NCH - 2), osem0).wait()
    pltpu.make_async_copy(rows1, out_ref(_NCH - 1), osem1).wait()


_sc_lookup = pl.kernel(
    _sc_body,
    out_type=[
        jax.ShapeDtypeStruct((_TP, _ROW), jnp.float32),
        jax.ShapeDtypeStruct((_TP,), jnp.int32),
    ],
    mesh=_mesh,
    scratch_types=[
        pltpu.VMEM((2 * _IPW,), jnp.int32),
        pltpu.VMEM((_IPW,), jnp.int32),
        pltpu.VMEM((_IPW,), jnp.int32),
        pltpu.VMEM((_IPW,), jnp.int32),
        pltpu.VMEM((2 * _PPW,), jnp.int32),
        pltpu.VMEM((_PPW,), jnp.int32),
        pltpu.VMEM((_PPW,), jnp.int32),
        pltpu.VMEM((81, _ROW), jnp.float32),
        pltpu.VMEM_SHARED((_NS * 81, _ROW), jnp.float32),
        pltpu.VMEM((_CH,), jnp.int32),
        pltpu.VMEM((_CH, _ROW), jnp.float32),
        pltpu.VMEM((_CH, _ROW), jnp.float32),
        pltpu.SemaphoreType.DMA,
        pltpu.SemaphoreType.DMA,
        pltpu.SemaphoreType.DMA,
    ],
    compiler_params=pltpu.CompilerParams(
        needs_layout_passes=False, disable_bounds_checks=True
    ),
)


def _tc_body(pidx_ref, ptab_ref, alias_ref, out_ref):
    del alias_ref
    ptab = ptab_ref[...]
    for r in range(_TCB // 128):
        pr = pidx_ref[r : r + 1, :]
        i81 = lax.broadcasted_iota(jnp.int32, (81, _ROW), 0)
        oht = (pr == i81).astype(jnp.float32)
        blk = lax.dot_general(
            oht, ptab, (((0,), (0,)), ((), ())),
            preferred_element_type=jnp.float32,
        )
        out_ref[pl.ds(r * 128, 128), :] = blk


_tc_expand = pl.pallas_call(
    _tc_body,
    grid=(_TCG,),
    in_specs=[
        pl.BlockSpec((_TCB // 128, 128), lambda b: (b, 0)),
        pl.BlockSpec((81, _ROW), lambda b: (0, 0)),
        pl.BlockSpec(memory_space=pltpu.HBM),
    ],
    out_specs=pl.BlockSpec((_TCB, _ROW), lambda b: (b, 0)),
    out_shape=jax.ShapeDtypeStruct((_TP, _ROW), jnp.float32),
    input_output_aliases={2: 0},
)


@jax.jit
def kernel(top_indices, distance_emb):
    emb = distance_emb.astype(jnp.float32)
    ptab = jnp.concatenate(
        [
            jnp.broadcast_to(emb[:, None, :], (9, 9, _EMB)),
            jnp.broadcast_to(emb[None, :, :], (9, 9, _EMB)),
        ],
        axis=-1,
    ).reshape(81, _ROW)
    top_flat = top_indices.reshape(-1).astype(jnp.int32)
    out_sc, pidx = _sc_lookup(
        top_flat, jnp.asarray(_WEF), jnp.asarray(_WOF), ptab
    )
    out = _tc_expand(pidx.reshape(_TP // 128, 128), ptab, out_sc)
    return out.reshape(_NWORDS, _K, _EMB)


# speedup vs baseline: 2.0021x; 1.0009x over previous
import functools
import numpy as np
import jax
import jax.numpy as jnp
from jax import lax
from jax.experimental import pallas as pl
from jax.experimental.pallas import tpu as pltpu
from jax.experimental.pallas import tpu_sc as plsc

_NWORDS = 8192
_K = 50
_EMB = 64
_ROW = 2 * _EMB
_TP = _NWORDS * _K // 2

_NC, _NS = 2, 16
_NWK = _NC * _NS
_IPW = _TP // _NWK
_PPW = 512
_CH = 128
_NCH = _PPW // _CH
_SCP = _NWK * _PPW
_SCBASE = _TP - _SCP

_TCB = 1024
_TCG = _SCBASE // _TCB

_WEF = (2 * np.arange(_TP, dtype=np.int32)) // _K
_WOF = (2 * np.arange(_TP, dtype=np.int32) + 1) // _K

_mesh = plsc.VectorSubcoreMesh(
    core_axis_name="c", subcore_axis_name="s", num_cores=_NC, num_subcores=_NS
)


def _bucket(word, top):
    d = jnp.maximum(word - top, 1)
    one = jnp.int32(1)
    zero = jnp.int32(0)
    bl = (
        4
        + jnp.where(d >= 8, one, zero)
        + jnp.where(d >= 16, one, zero)
        + jnp.where(d >= 32, one, zero)
        + jnp.where(d >= 64, one, zero)
    )
    return jnp.where(d < 5, d - 1, bl)


def _sc_body(top_hbm, we_hbm, wo_hbm, ptab_hbm, out_hbm, pidx_hbm,
             top_v, we_v, wo_v, pidx_v, ttop_v, twe_v, two_v,
             ptab_v, ptab_s, idx_v, rows0, rows1, gsem, osem0, osem1):
    sid = lax.axis_index("s")
    wid = sid * _NC + lax.axis_index("c")
    lane = lax.iota(jnp.int32, 16)
    lane2 = 2 * lane

    ibase = wid * _IPW
    pltpu.sync_copy(top_hbm.at[pl.ds(2 * ibase, 2 * _IPW)], top_v)
    pltpu.sync_copy(we_hbm.at[pl.ds(ibase, _IPW)], we_v)
    pltpu.sync_copy(wo_hbm.at[pl.ds(ibase, _IPW)], wo_v)

    def idx_group(g, carry):
        j = g * 16
        te = plsc.load_gather(top_v, [2 * j + lane2])
        to = plsc.load_gather(top_v, [2 * j + lane2 + 1])
        we = we_v[pl.ds(j, 16)]
        wo = wo_v[pl.ds(j, 16)]
        pidx_v[pl.ds(j, 16)] = _bucket(we, te) * 9 + _bucket(wo, to)
        return carry

    lax.fori_loop(0, _IPW // 16, idx_group, 0)
    pltpu.sync_copy(pidx_v, pidx_hbm.at[pl.ds(ibase, _IPW)])

    pbase = _SCBASE + wid * _PPW
    pltpu.sync_copy(top_hbm.at[pl.ds(2 * pbase, 2 * _PPW)], ttop_v)
    pltpu.sync_copy(we_hbm.at[pl.ds(pbase, _PPW)], twe_v)
    pltpu.sync_copy(wo_hbm.at[pl.ds(pbase, _PPW)], two_v)
    pltpu.sync_copy(ptab_hbm, ptab_v)
    pltpu.sync_copy(ptab_hbm, ptab_s.at[pl.ds(sid * 81, 81)])
    sbase = sid * 81

    def pair_indices(j):
        te = plsc.load_gather(ttop_v, [2 * j + lane2])
        to = plsc.load_gather(ttop_v, [2 * j + lane2 + 1])
        we = twe_v[pl.ds(j, 16)]
        wo = two_v[pl.ds(j, 16)]
        return _bucket(we, te) * 9 + _bucket(wo, to)

    def compute_idx(c, idxbuf):
        for g in range(_CH // 16):
            idxbuf[pl.ds(g * 16, 16)] = pair_indices(c * _CH + g * 16) + sbase

    def fill(c, rowsbuf):
        def group(g, carry):
            gaddr = pair_indices(c * _CH + g * 16)
            prow = lane + g * 16

            @plsc.parallel_loop(0, _ROW, step=1, unroll=8)
            def colloop(t):
                colv = (t + lane) & (_ROW - 1)
                v = plsc.load_gather(ptab_v, [gaddr, colv])
                plsc.store_scatter(rowsbuf, [prow, colv], v)

            return carry

        lax.fori_loop(0, _CH // 16, group, 0)

    def out_ref(c):
        return out_hbm.at[pl.ds(pbase + c * _CH, _CH)]

    def step(i, carry):
        c0 = 2 * i
        c1 = 2 * i + 1
        compute_idx(c0, idx_v)

        @pl.when(i >= 1)
        def _():
            pltpu.make_async_copy(rows0, out_ref(c0 - 2), osem0).wait()

        pltpu.async_copy(ptab_s.at[idx_v], rows0, gsem)

        @pl.when(i >= 1)
        def _():
            pltpu.make_async_copy(rows1, out_ref(c1 - 2), osem1).wait()

        fill(c1, rows1)
        pltpu.async_copy(rows1, out_ref(c1), osem1)
        pltpu.make_async_copy(ptab_s.at[idx_v], rows0, gsem).wait()
        pltpu.async_copy(rows0, out_ref(c0), osem0)
        return carry

    lax.fori_loop(0, _NCH // 2, step, 0)
    pltpu.make_async_copy(rows0, out_ref(_NCH - 2), osem0).wait()
    pltpu.make_async_copy(rows1, out_ref(_NCH - 1), osem1).wait()


_sc_lookup = pl.kernel(
    _sc_body,
    out_type=[
        jax.ShapeDtypeStruct((_TP, _ROW), jnp.float32),
        jax.ShapeDtypeStruct((_TP,), jnp.int32),
    ],
    mesh=_mesh,
    scratch_types=[
        pltpu.VMEM((2 * _IPW,), jnp.int32),
        pltpu.VMEM((_IPW,), jnp.int32),
        pltpu.VMEM((_IPW,), jnp.int32),
        pltpu.VMEM((_IPW,), jnp.int32),
        pltpu.VMEM((2 * _PPW,), jnp.int32),
        pltpu.VMEM((_PPW,), jnp.int32),
        pltpu.VMEM((_PPW,), jnp.int32),
        pltpu.VMEM((81, _ROW), jnp.float32),
        pltpu.VMEM_SHARED((_NS * 81, _ROW), jnp.float32),
        pltpu.VMEM((_CH,), jnp.int32),
        pltpu.VMEM((_CH, _ROW), jnp.float32),
        pltpu.VMEM((_CH, _ROW), jnp.float32),
        pltpu.SemaphoreType.DMA,
        pltpu.SemaphoreType.DMA,
        pltpu.SemaphoreType.DMA,
    ],
    compiler_params=pltpu.CompilerParams(
        needs_layout_passes=False, disable_bounds_checks=True
    ),
)


def _tc_body(pidx_ref, ptab_ref, alias_ref, top_ref, out_ref):
    del alias_ref, top_ref
    i81 = lax.broadcasted_iota(jnp.int32, (81, 128), 0)
    ohts = []
    for r in range(_TCB // 128):
        pr = pidx_ref[r : r + 1, :]
        ohts.append((pr == i81).astype(jnp.float32))
    oht = jnp.concatenate(ohts, axis=1)
    out_ref[...] = lax.dot_general(
        oht, ptab_ref[...], (((0,), (0,)), ((), ())),
        preferred_element_type=jnp.float32,
    )


_tc_expand = pl.pallas_call(
    _tc_body,
    grid=(_TCG,),
    in_specs=[
        pl.BlockSpec((_TCB // 128, 128), lambda b: (b, 0)),
        pl.BlockSpec((81, _ROW), lambda b: (0, 0)),
        pl.BlockSpec(memory_space=pltpu.HBM),
        pl.BlockSpec(memory_space=pltpu.HBM),
    ],
    out_specs=pl.BlockSpec((_TCB, _ROW), lambda b: (b, 0)),
    out_shape=jax.ShapeDtypeStruct((_TP, _ROW), jnp.float32),
    input_output_aliases={2: 0},
)


@jax.jit
def kernel(top_indices, distance_emb):
    emb = distance_emb.astype(jnp.float32)
    ptab = jnp.concatenate(
        [
            jnp.broadcast_to(emb[:, None, :], (9, 9, _EMB)),
            jnp.broadcast_to(emb[None, :, :], (9, 9, _EMB)),
        ],
        axis=-1,
    ).reshape(81, _ROW)
    top_flat = top_indices.reshape(-1).astype(jnp.int32)
    out_sc, pidx = _sc_lookup(
        top_flat, jnp.asarray(_WEF), jnp.asarray(_WOF), ptab
    )
    out = _tc_expand(pidx.reshape(_TP // 128, 128), ptab, out_sc, top_flat)
    return out.reshape(_NWORDS, _K, _EMB)
